# baseline (speedup 1.0000x reference)
import functools

import jax
import jax.numpy as jnp
from jax import lax
from jax.experimental import pallas as pl
from jax.experimental.pallas import tpu as pltpu
from jax.experimental.pallas import tpu_sc as plsc

NC = 2
NS = 16
NW = NC * NS
L = 16
D = 128
CEDGE = 128


def _fill(buf, rows, value):
  v = jnp.full((L,), value, jnp.float32)

  def body(i, _):
    for j in range(D // L):
      buf[i, pl.ds(j * L, L)] = v
    return 0

  lax.fori_loop(0, rows, body, 0)


def _make_step(n_pad, ch):
  rpt = n_pad // NS

  mesh = plsc.VectorSubcoreMesh(core_axis_name="c", subcore_axis_name="s")

  assert ch % 2 == 0
  half = ch // 2

  @functools.partial(
      pl.kernel,
      mesh=mesh,
      out_type=jax.ShapeDtypeStruct((NC, n_pad, D), jnp.float32),
      scratch_types=[
          pltpu.VMEM((half, CEDGE), jnp.int32),
          pltpu.VMEM((half, CEDGE), jnp.int32),
          pltpu.VMEM((CEDGE, D), jnp.float32),
          pltpu.VMEM_SHARED((n_pad, D), jnp.float32),
          pltpu.SemaphoreType.DMA,
      ],
  )
  def step(y_hbm, colx_hbm, rowx_hbm, out_hbm, cidx, ridx, gbuf, acc, sem):
    c = lax.axis_index("c")
    s = lax.axis_index("s")
    wid = s * NC + c
    base = s * rpt

    _fill(gbuf, CEDGE, 0.0)
    for k in range(rpt // CEDGE):
      pltpu.sync_copy(gbuf, acc.at[pl.ds(base + k * CEDGE, CEDGE)])
    plsc.subcore_barrier()

    for off in range(2):
      pltpu.sync_copy(colx_hbm.at[wid * 2 + off], cidx)
      pltpu.sync_copy(rowx_hbm.at[wid * 2 + off], ridx)

      def body(j, _):
        pltpu.async_copy(y_hbm.at[cidx.at[j]], gbuf, sem).wait()
        pltpu.sync_copy(gbuf, acc.at[ridx.at[j]], add=True)
        return 0

      lax.fori_loop(0, half, body, 0)

    plsc.subcore_barrier()

    pltpu.sync_copy(acc.at[pl.ds(base, rpt)], out_hbm.at[c, pl.ds(base, rpt)])

  return step


def _make_deg(n_pad, ch):
  rpt = n_pad // NS
  mesh = plsc.VectorSubcoreMesh(core_axis_name="c", subcore_axis_name="s")

  half = ch // 2

  @functools.partial(
      pl.kernel,
      mesh=mesh,
      out_type=jax.ShapeDtypeStruct((NC, n_pad, D), jnp.float32),
      scratch_types=[
          pltpu.VMEM((half, CEDGE), jnp.int32),
          pltpu.VMEM((CEDGE, D), jnp.float32),
          pltpu.VMEM_SHARED((n_pad, D), jnp.float32),
      ],
  )
  def deg(colx_hbm, out_hbm, cidx, gbuf, acc):
    c = lax.axis_index("c")
    s = lax.axis_index("s")
    wid = s * NC + c
    base = s * rpt

    _fill(gbuf, CEDGE, 0.0)
    for k in range(rpt // CEDGE):
      pltpu.sync_copy(gbuf, acc.at[pl.ds(base + k * CEDGE, CEDGE)])
    plsc.subcore_barrier()

    _fill(gbuf, CEDGE, 1.0)

    def chunk(j, _):
      pltpu.sync_copy(gbuf, acc.at[cidx.at[j]], add=True)
      return 0

    for off in range(2):
      pltpu.sync_copy(colx_hbm.at[wid * 2 + off], cidx)
      lax.fori_loop(0, half, chunk, 0)
    plsc.subcore_barrier()

    pltpu.sync_copy(acc.at[pl.ds(base, rpt)], out_hbm.at[c, pl.ds(base, rpt)])

  return deg


def _combine(p, a, b, block_rows):
  n_pad = a.shape[0]

  def body(p_ref, a_ref, b_ref, o_ref):
    o_ref[...] = a_ref[...] + b_ref[...] * (p_ref[0] + p_ref[1])

  return pl.pallas_call(
      body,
      grid=(n_pad // block_rows,),
      in_specs=[
          pl.BlockSpec((2, block_rows, D), lambda i: (0, i, 0)),
          pl.BlockSpec((block_rows, D), lambda i: (i, 0)),
          pl.BlockSpec((block_rows, D), lambda i: (i, 0)),
      ],
      out_specs=pl.BlockSpec((block_rows, D), lambda i: (i, 0)),
      out_shape=jax.ShapeDtypeStruct((n_pad, D), jnp.float32),
  )(p, a, b)


def _constants(degp, x_pad, m_pad, block_rows):
  n_pad = x_pad.shape[0]

  def body(dp_ref, x_ref, m_ref, c0_ref, c1_ref, f0_ref, f1_ref):
    deg = dp_ref[0] + dp_ref[1]
    dinv = jnp.where(deg > 0.0, lax.rsqrt(deg), 0.0)
    m = m_ref[...]
    xv = x_ref[...]
    c0_ref[...] = m * dinv * xv
    c1_ref[...] = (1.0 - m) * dinv * dinv
    f0_ref[...] = m * xv
    f1_ref[...] = (1.0 - m) * dinv

  shp = jax.ShapeDtypeStruct((n_pad, D), jnp.float32)
  return pl.pallas_call(
      body,
      grid=(n_pad // block_rows,),
      in_specs=[
          pl.BlockSpec((2, block_rows, D), lambda i: (0, i, 0)),
          pl.BlockSpec((block_rows, D), lambda i: (i, 0)),
          pl.BlockSpec((block_rows, D), lambda i: (i, 0)),
      ],
      out_specs=[pl.BlockSpec((block_rows, D), lambda i: (i, 0))] * 4,
      out_shape=[shp, shp, shp, shp],
  )(degp, x_pad, m_pad)


def kernel(x, edge_index, mask):
  n, d = x.shape
  assert d == D
  e = edge_index.shape[1]
  num_iters = 40

  n_pad = ((n + NS * CEDGE - 1) // (NS * CEDGE)) * (NS * CEDGE)
  ch = (e + NW * CEDGE - 1) // (NW * CEDGE)
  ch = ((ch + 3) // 4) * 4
  e_pad = NW * ch * CEDGE

  row = edge_index[0].astype(jnp.int32)
  col = edge_index[1].astype(jnp.int32)
  pad = jnp.full((e_pad - e,), n, jnp.int32)
  colx = jnp.concatenate([col, pad]).reshape(NW * 2, ch // 2, CEDGE)
  rowx = jnp.concatenate([row, pad]).reshape(NW * 2, ch // 2, CEDGE)

  x_pad = jnp.pad(x, ((0, n_pad - n), (0, 0)))
  m_pad = jnp.pad(
      jnp.broadcast_to(mask.astype(jnp.float32)[:, None], (n, D)),
      ((0, n_pad - n), (0, 0)))

  step = _make_step(n_pad, ch)
  degk = _make_deg(n_pad, ch)
  block_rows = n_pad // 8

  degp = degk(colx)
  c0, c1, f0, f1 = _constants(degp, x_pad, m_pad, block_rows)

  def body(_, y):
    p = step(y, colx, rowx)
    return _combine(p, c0, c1, block_rows)

  y = lax.fori_loop(0, num_iters - 1, body, c0)
  p = step(y, colx, rowx)
  out = _combine(p, f0, f1, block_rows)
  return out[:n]


# speedup vs baseline: 1.4747x; 1.4747x over previous
import functools

import jax
import jax.numpy as jnp
from jax import lax
from jax.experimental import pallas as pl
from jax.experimental.pallas import tpu as pltpu
from jax.experimental.pallas import tpu_sc as plsc

NC = 2
NS = 16
NW = NC * NS
L = 16
D = 128
CEDGE = 128


def _fill(buf, rows, value):
  v = jnp.full((L,), value, jnp.float32)

  def body(i, _):
    for j in range(D // L):
      buf[i, pl.ds(j * L, L)] = v
    return 0

  lax.fori_loop(0, rows, body, 0)


def _make_step(n_pad, ch):
  rpt = n_pad // NS

  mesh = plsc.VectorSubcoreMesh(core_axis_name="c", subcore_axis_name="s")

  @functools.partial(
      pl.kernel,
      mesh=mesh,
      out_type=jax.ShapeDtypeStruct((NC, n_pad, D), jnp.float32),
      scratch_types=[
          pltpu.VMEM((ch, CEDGE), jnp.int32),
          pltpu.VMEM((ch, CEDGE), jnp.int32),
          pltpu.VMEM((CEDGE, D), jnp.float32),
          pltpu.VMEM_SHARED((n_pad, D), jnp.float32),
          pltpu.SemaphoreType.DMA,
      ],
  )
  def step(y_hbm, colx_hbm, rowx_hbm, out_hbm, cidx, ridx, gbuf, acc, sem):
    c = lax.axis_index("c")
    s = lax.axis_index("s")
    wid = s * NC + c
    base = s * rpt

    _fill(gbuf, CEDGE, 0.0)
    for k in range(rpt // CEDGE):
      pltpu.sync_copy(gbuf, acc.at[pl.ds(base + k * CEDGE, CEDGE)])
    plsc.subcore_barrier()

    pltpu.sync_copy(colx_hbm.at[wid], cidx)
    pltpu.sync_copy(rowx_hbm.at[wid], ridx)

    def chunk(j, _):
      pltpu.async_copy(y_hbm.at[cidx.at[j]], gbuf, sem).wait()
      pltpu.sync_copy(gbuf, acc.at[ridx.at[j]], add=True)
      return 0

    lax.fori_loop(0, ch, chunk, 0)
    plsc.subcore_barrier()

    pltpu.sync_copy(acc.at[pl.ds(base, rpt)], out_hbm.at[c, pl.ds(base, rpt)])

  return step


def _make_deg(n_pad, ch):
  rpt = n_pad // NS
  mesh = plsc.VectorSubcoreMesh(core_axis_name="c", subcore_axis_name="s")

  @functools.partial(
      pl.kernel,
      mesh=mesh,
      out_type=jax.ShapeDtypeStruct((NC, n_pad, D), jnp.float32),
      scratch_types=[
          pltpu.VMEM((ch, CEDGE), jnp.int32),
          pltpu.VMEM((CEDGE, D), jnp.float32),
          pltpu.VMEM_SHARED((n_pad, D), jnp.float32),
      ],
  )
  def deg(colx_hbm, out_hbm, cidx, gbuf, acc):
    c = lax.axis_index("c")
    s = lax.axis_index("s")
    wid = s * NC + c
    base = s * rpt

    _fill(gbuf, CEDGE, 0.0)
    for k in range(rpt // CEDGE):
      pltpu.sync_copy(gbuf, acc.at[pl.ds(base + k * CEDGE, CEDGE)])
    plsc.subcore_barrier()

    pltpu.sync_copy(colx_hbm.at[wid], cidx)
    _fill(gbuf, CEDGE, 1.0)

    def chunk(j, _):
      pltpu.sync_copy(gbuf, acc.at[cidx.at[j]], add=True)
      return 0

    lax.fori_loop(0, ch, chunk, 0)
    plsc.subcore_barrier()

    pltpu.sync_copy(acc.at[pl.ds(base, rpt)], out_hbm.at[c, pl.ds(base, rpt)])

  return deg


def _combine(p, a, b, block_rows):
  n_pad = a.shape[0]

  def body(p_ref, a_ref, b_ref, o_ref):
    o_ref[...] = a_ref[...] + b_ref[...] * (p_ref[0] + p_ref[1])

  return pl.pallas_call(
      body,
      grid=(n_pad // block_rows,),
      in_specs=[
          pl.BlockSpec((2, block_rows, D), lambda i: (0, i, 0)),
          pl.BlockSpec((block_rows, D), lambda i: (i, 0)),
          pl.BlockSpec((block_rows, D), lambda i: (i, 0)),
      ],
      out_specs=pl.BlockSpec((block_rows, D), lambda i: (i, 0)),
      out_shape=jax.ShapeDtypeStruct((n_pad, D), jnp.float32),
  )(p, a, b)


def _constants(degp, x_pad, m_pad, block_rows):
  n_pad = x_pad.shape[0]

  def body(dp_ref, x_ref, m_ref, c0_ref, c1_ref, f0_ref, f1_ref):
    deg = dp_ref[0] + dp_ref[1]
    dinv = jnp.where(deg > 0.0, lax.rsqrt(deg), 0.0)
    m = m_ref[...]
    xv = x_ref[...]
    c0_ref[...] = m * dinv * xv
    c1_ref[...] = (1.0 - m) * dinv * dinv
    f0_ref[...] = m * xv
    f1_ref[...] = (1.0 - m) * dinv

  shp = jax.ShapeDtypeStruct((n_pad, D), jnp.float32)
  return pl.pallas_call(
      body,
      grid=(n_pad // block_rows,),
      in_specs=[
          pl.BlockSpec((2, block_rows, D), lambda i: (0, i, 0)),
          pl.BlockSpec((block_rows, D), lambda i: (i, 0)),
          pl.BlockSpec((block_rows, D), lambda i: (i, 0)),
      ],
      out_specs=[pl.BlockSpec((block_rows, D), lambda i: (i, 0))] * 4,
      out_shape=[shp, shp, shp, shp],
  )(degp, x_pad, m_pad)


def kernel(x, edge_index, mask):
  n, d = x.shape
  assert d == D
  e = edge_index.shape[1]
  num_iters = 40

  n_pad = ((n + NS * CEDGE - 1) // (NS * CEDGE)) * (NS * CEDGE)
  ch = (e + NW * CEDGE - 1) // (NW * CEDGE)
  e_pad = NW * ch * CEDGE

  row = edge_index[0].astype(jnp.int32)
  col = edge_index[1].astype(jnp.int32)
  pad = jnp.full((e_pad - e,), n, jnp.int32)
  colx = jnp.concatenate([col, pad]).reshape(NW, ch, CEDGE)
  rowx = jnp.concatenate([row, pad]).reshape(NW, ch, CEDGE)

  x_pad = jnp.pad(x, ((0, n_pad - n), (0, 0)))
  m_pad = jnp.pad(
      jnp.broadcast_to(mask.astype(jnp.float32)[:, None], (n, D)),
      ((0, n_pad - n), (0, 0)))

  step = _make_step(n_pad, ch)
  degk = _make_deg(n_pad, ch)
  block_rows = n_pad // 8

  degp = degk(colx)
  c0, c1, f0, f1 = _constants(degp, x_pad, m_pad, block_rows)

  def body(_, y):
    p = step(y, colx, rowx)
    return _combine(p, c0, c1, block_rows)

  y = lax.fori_loop(0, num_iters - 1, body, c0)
  p = step(y, colx, rowx)
  out = _combine(p, f0, f1, block_rows)
  return out[:n]
